# baseline (speedup 1.0000x reference)
import functools

import jax
import jax.numpy as jnp
from jax import lax
from jax.experimental import pallas as pl
from jax.experimental.pallas import tpu as pltpu
from jax.experimental.pallas import tpu_sc as plsc

B, L, H, W, C = 2, 4, 128, 128, 256
N = 1000
CS = 7
M = B * N * CS * CS
LANES = 16
NC, NS = 2, 16
NW = NC * NS
CH = 16
TOT_CH = M // CH
CPW = -(-TOT_CH // NW)
ROWS_W = CPW * CH
M_PAD = NW * ROWS_W
HC = C // 128
G = 4 * HC

_STRIDES = (4.0, 8.0, 16.0, 32.0)


def _sample_grid(rois):
    x1, y1, x2, y2 = rois[..., 0], rois[..., 1], rois[..., 2], rois[..., 3]
    roi_area = (y2 - y1) * (x2 - x1)
    lv = (jnp.log(jnp.sqrt(jnp.maximum(roi_area, 1e-12)) / 224.0)
          / jnp.log(2.0) + 4.0)
    li = jnp.clip(jnp.round(lv).astype(jnp.int32), 2, 5) - 2
    strides = jnp.asarray(_STRIDES, dtype=jnp.float32)[li]
    Hf = jnp.float32(H)
    Wf = jnp.float32(W)
    nx1 = x1 / strides / Wf
    ny1 = y1 / strides / Hf
    nx2 = x2 / strides / Wf
    ny2 = y2 / strides / Hf
    i = jnp.arange(CS, dtype=jnp.float32)
    ys = (ny1[..., None] * (Hf - 1.0)
          + i[None, None, :] * ((ny2 - ny1)[..., None] * (Hf - 1.0) / (CS - 1)))
    xs = (nx1[..., None] * (Wf - 1.0)
          + i[None, None, :] * ((nx2 - nx1)[..., None] * (Wf - 1.0) / (CS - 1)))
    yy = jnp.broadcast_to(ys[:, :, :, None], (B, N, CS, CS))
    xx = jnp.broadcast_to(xs[:, :, None, :], (B, N, CS, CS))
    yy = yy.reshape(B, N, CS * CS)
    xx = xx.reshape(B, N, CS * CS)
    return yy, xx, li[:, :, None]


def _prep_body(yy_ref, xx_ref, li_ref, idx_ref, w_ref):
    Hf = jnp.float32(H)
    Wf = jnp.float32(W)
    for b in range(B):
        yy = yy_ref[b]
        xx = xx_ref[b]
        li = li_ref[b]
        valid = (yy >= 0.0) & (yy <= Hf - 1.0) & (xx >= 0.0) & (xx <= Wf - 1.0)
        y0 = jnp.floor(yy)
        x0 = jnp.floor(xx)
        wy = yy - y0
        wx = xx - x0
        y0i = jnp.clip(y0.astype(jnp.int32), 0, H - 1)
        x0i = jnp.clip(x0.astype(jnp.int32), 0, W - 1)
        y1i = jnp.minimum(y0i + 1, H - 1)
        x1i = jnp.minimum(x0i + 1, W - 1)
        base = li * (H * W) + b * (L * H * W)
        r0 = base + y0i * W
        r1 = base + y1i * W
        vf = valid.astype(jnp.float32)
        wx0 = 1.0 - wx
        wy0 = 1.0 - wy
        for k, rid in enumerate((r0 + x0i, r0 + x1i, r1 + x0i, r1 + x1i)):
            idx_ref[b, 2 * k] = rid
            idx_ref[b, 2 * k + 1] = rid + B * L * H * W
        w_ref[b, 0] = wx0 * wy0 * vf
        w_ref[b, 1] = wx * wy0 * vf
        w_ref[b, 2] = wx0 * wy * vf
        w_ref[b, 3] = wx * wy * vf


_prep_call = pl.pallas_call(
    _prep_body,
    out_shape=(
        jax.ShapeDtypeStruct((B, G, N, CS * CS), jnp.int32),
        jax.ShapeDtypeStruct((B, 4, N, CS * CS), jnp.float32),
    ),
)


def _relayout_body(in_ref, out_ref):
    out_ref[...] = in_ref[0, 0].reshape(8 * W, 128)


_relayout_call = pl.pallas_call(
    _relayout_body,
    grid=(B, L, H // 8, HC),
    in_specs=[pl.BlockSpec((1, 1, 8, W, 128),
                           lambda b, l, hb, j: (b, l, hb, 0, j))],
    out_specs=pl.BlockSpec(
        (8 * W, 128),
        lambda b, l, hb, j: (j * (B * L * H // 8) + (b * L + l) * (H // 8)
                             + hb, 0)),
    out_shape=jax.ShapeDtypeStruct((B * L * H * W * HC, 128), jnp.float32),
)


def _sc_body(table, idxf, wf, out, idx_v, w_v,
             gbuf0, gbuf1, obuf, gsem0, gsem1):
    cid = lax.axis_index("c")
    sid = lax.axis_index("s")
    wid = sid * NC + cid
    base = wid * ROWS_W
    nch = jnp.minimum(CPW, TOT_CH - CPW * wid)
    for k in range(G):
        pltpu.sync_copy(idxf.at[pl.ds(k * M_PAD + base, ROWS_W)],
                        idx_v.at[k])
    for k in range(4):
        pltpu.sync_copy(wf.at[pl.ds(k * M_PAD + base, ROWS_W)],
                        w_v.at[pl.ds(k * ROWS_W, ROWS_W)])

    gbufs = (gbuf0, gbuf1)
    gsems = (gsem0, gsem1)

    def gather_copies(c, b):
        return [pltpu.make_async_copy(
            table.at[idx_v.at[k, pl.ds(c * CH, CH)]],
            gbufs[b].at[pl.ds(k * CH, CH)], gsems[b]) for k in range(G)]

    @pl.when(nch > 0)
    def _():
        for cp in gather_copies(0, 0):
            cp.start()

    def pair(g, carry):
        for b in range(2):
            c = 2 * g + b

            @pl.when(c < nch)
            def _(c=c, b=b):
                @pl.when(c + 1 < nch)
                def _():
                    for cp in gather_copies(c + 1, 1 - b):
                        cp.start()

                gbuf = gbufs[b]
                for cp in gather_copies(c, b):
                    cp.wait()

                wchunk = [w_v[pl.ds(k * ROWS_W + c * CH, LANES)]
                          for k in range(4)]

                def row(r, carry2):
                    lane = jnp.full((LANES,), r, jnp.int32)
                    wvs = [wc.at[lane].get(mode="promise_in_bounds")
                           for wc in wchunk]

                    def vec(v, carry3):
                        for j in range(HC):
                            col = pl.ds(v * LANES, LANES)
                            acc = gbuf[j * CH + r, col] * wvs[0]
                            for k in range(1, 4):
                                acc = acc + (gbuf[(2 * k + j) * CH + r, col]
                                             * wvs[k])
                            obuf[r, pl.ds(j * 128 + v * LANES, LANES)] = acc
                        return carry3

                    lax.fori_loop(0, 128 // LANES, vec, 0)
                    return carry2

                lax.fori_loop(0, CH, row, 0)
                pltpu.sync_copy(obuf, out.at[pl.ds(base + c * CH, CH)])
        return carry

    lax.fori_loop(0, (CPW + 1) // 2, pair, 0)


@functools.cache
def _sc_call():
    return pl.kernel(
        _sc_body,
        out_type=jax.ShapeDtypeStruct((M, C), jnp.float32),
        mesh=plsc.VectorSubcoreMesh(core_axis_name="c", subcore_axis_name="s"),
        scratch_types=[
            pltpu.VMEM((G, ROWS_W), jnp.int32),
            pltpu.VMEM((4 * ROWS_W,), jnp.float32),
            pltpu.VMEM((CH * G, 128), jnp.float32),
            pltpu.VMEM((CH * G, 128), jnp.float32),
            pltpu.VMEM((CH, C), jnp.float32),
            pltpu.SemaphoreType.DMA,
            pltpu.SemaphoreType.DMA,
        ],
    )


def kernel(feature_maps, rois):
    yy, xx, li = _sample_grid(rois)
    idx_p, w_p = _prep_call(yy, xx, li)
    idx_t = idx_p.transpose(1, 0, 2, 3).reshape(G, M)
    w_t = w_p.transpose(1, 0, 2, 3).reshape(4, M)
    idx_f = jnp.pad(idx_t, ((0, 0), (0, M_PAD - M))).reshape(-1)
    w_f = jnp.pad(w_t, ((0, 0), (0, M_PAD - M))).reshape(-1)
    table = _relayout_call(feature_maps)
    out = _sc_call()(table, idx_f, w_f)
    return out.reshape(B, N, CS, CS, C)


# speedup vs baseline: 1.2810x; 1.2810x over previous
import functools

import jax
import jax.numpy as jnp
from jax import lax
from jax.experimental import pallas as pl
from jax.experimental.pallas import tpu as pltpu
from jax.experimental.pallas import tpu_sc as plsc

B, L, H, W, C = 2, 4, 128, 128, 256
N = 1000
CS = 7
M = B * N * CS * CS
LANES = 16
NC, NS = 2, 16
NW = NC * NS
CH = 16
TOT_CH = M // CH
CPW = -(-TOT_CH // NW)
ROWS_W = CPW * CH
M_PAD = NW * ROWS_W
HC = C // 128
G = 4 * HC

_STRIDES = (4.0, 8.0, 16.0, 32.0)


def _sample_grid(rois):
    x1, y1, x2, y2 = rois[..., 0], rois[..., 1], rois[..., 2], rois[..., 3]
    roi_area = (y2 - y1) * (x2 - x1)
    lv = (jnp.log(jnp.sqrt(jnp.maximum(roi_area, 1e-12)) / 224.0)
          / jnp.log(2.0) + 4.0)
    li = jnp.clip(jnp.round(lv).astype(jnp.int32), 2, 5) - 2
    strides = jnp.asarray(_STRIDES, dtype=jnp.float32)[li]
    Hf = jnp.float32(H)
    Wf = jnp.float32(W)
    nx1 = x1 / strides / Wf
    ny1 = y1 / strides / Hf
    nx2 = x2 / strides / Wf
    ny2 = y2 / strides / Hf
    i = jnp.arange(CS, dtype=jnp.float32)
    ys = (ny1[..., None] * (Hf - 1.0)
          + i[None, None, :] * ((ny2 - ny1)[..., None] * (Hf - 1.0) / (CS - 1)))
    xs = (nx1[..., None] * (Wf - 1.0)
          + i[None, None, :] * ((nx2 - nx1)[..., None] * (Wf - 1.0) / (CS - 1)))
    yy = jnp.broadcast_to(ys[:, :, :, None], (B, N, CS, CS))
    xx = jnp.broadcast_to(xs[:, :, None, :], (B, N, CS, CS))
    yy = yy.reshape(B, N, CS * CS)
    xx = xx.reshape(B, N, CS * CS)
    return yy, xx, li[:, :, None]


def _prep_body(yy_ref, xx_ref, li_ref, idx_ref, w_ref):
    Hf = jnp.float32(H)
    Wf = jnp.float32(W)
    for b in range(B):
        yy = yy_ref[b]
        xx = xx_ref[b]
        li = li_ref[b]
        valid = (yy >= 0.0) & (yy <= Hf - 1.0) & (xx >= 0.0) & (xx <= Wf - 1.0)
        y0 = jnp.floor(yy)
        x0 = jnp.floor(xx)
        wy = yy - y0
        wx = xx - x0
        y0i = jnp.clip(y0.astype(jnp.int32), 0, H - 1)
        x0i = jnp.clip(x0.astype(jnp.int32), 0, W - 1)
        y1i = jnp.minimum(y0i + 1, H - 1)
        x1i = jnp.minimum(x0i + 1, W - 1)
        base = li * (H * W) + b * (L * H * W)
        r0 = base + y0i * W
        r1 = base + y1i * W
        vf = valid.astype(jnp.float32)
        wx0 = 1.0 - wx
        wy0 = 1.0 - wy
        for k, rid in enumerate((r0 + x0i, r0 + x1i, r1 + x0i, r1 + x1i)):
            idx_ref[b, 2 * k] = 2 * rid
            idx_ref[b, 2 * k + 1] = 2 * rid + 1
        w_ref[b, 0] = wx0 * wy0 * vf
        w_ref[b, 1] = wx * wy0 * vf
        w_ref[b, 2] = wx0 * wy * vf
        w_ref[b, 3] = wx * wy * vf


_prep_call = pl.pallas_call(
    _prep_body,
    out_shape=(
        jax.ShapeDtypeStruct((B, G, N, CS * CS), jnp.int32),
        jax.ShapeDtypeStruct((B, 4, N, CS * CS), jnp.float32),
    ),
)


NB = 8


def _repack_body(in_ref, out_ref):
    out_ref[...] = in_ref[...].reshape(1, NB, CS, CS, C)


_repack_call = pl.pallas_call(
    _repack_body,
    grid=(B, N // NB),
    in_specs=[pl.BlockSpec((NB * CS * CS, C),
                           lambda b, nb: (b * (N // NB) + nb, 0))],
    out_specs=pl.BlockSpec((1, NB, CS, CS, C),
                           lambda b, nb: (b, nb, 0, 0, 0)),
    out_shape=jax.ShapeDtypeStruct((B, N, CS, CS, C), jnp.float32),
)


def _sc_body(table, idxf, wf, out, idx_v, w_v,
             gbuf0, gbuf1, obuf, gsem0, gsem1):
    cid = lax.axis_index("c")
    sid = lax.axis_index("s")
    wid = sid * NC + cid
    base = wid * ROWS_W
    nch = jnp.minimum(CPW, TOT_CH - CPW * wid)
    for k in range(G):
        pltpu.sync_copy(idxf.at[pl.ds(k * M_PAD + base, ROWS_W)],
                        idx_v.at[k])
    for k in range(4):
        pltpu.sync_copy(wf.at[pl.ds(k * M_PAD + base, ROWS_W)],
                        w_v.at[pl.ds(k * ROWS_W, ROWS_W)])

    gbufs = (gbuf0, gbuf1)
    gsems = (gsem0, gsem1)

    def gather_copies(c, b):
        return [pltpu.make_async_copy(
            table.at[idx_v.at[k, pl.ds(c * CH, CH)]],
            gbufs[b].at[pl.ds(k * CH, CH)], gsems[b]) for k in range(G)]

    @pl.when(nch > 0)
    def _():
        for cp in gather_copies(0, 0):
            cp.start()

    def pair(g, carry):
        for b in range(2):
            c = 2 * g + b

            @pl.when(c < nch)
            def _(c=c, b=b):
                @pl.when(c + 1 < nch)
                def _():
                    for cp in gather_copies(c + 1, 1 - b):
                        cp.start()

                gbuf = gbufs[b]
                for cp in gather_copies(c, b):
                    cp.wait()

                wchunk = [w_v[pl.ds(k * ROWS_W + c * CH, LANES)]
                          for k in range(4)]

                def row(r, carry2):
                    lane = jnp.full((LANES,), r, jnp.int32)
                    wvs = [wc.at[lane].get(mode="promise_in_bounds")
                           for wc in wchunk]

                    def vec(v, carry3):
                        for j in range(HC):
                            col = pl.ds(v * LANES, LANES)
                            acc = gbuf[j * CH + r, col] * wvs[0]
                            for k in range(1, 4):
                                acc = acc + (gbuf[(2 * k + j) * CH + r, col]
                                             * wvs[k])
                            obuf[r, pl.ds(j * 128 + v * LANES, LANES)] = acc
                        return carry3

                    lax.fori_loop(0, 128 // LANES, vec, 0)
                    return carry2

                lax.fori_loop(0, CH, row, 0)
                pltpu.sync_copy(obuf, out.at[pl.ds(base + c * CH, CH)])
        return carry

    lax.fori_loop(0, (CPW + 1) // 2, pair, 0)


@functools.cache
def _sc_call():
    return pl.kernel(
        _sc_body,
        out_type=jax.ShapeDtypeStruct((M, C), jnp.float32),
        mesh=plsc.VectorSubcoreMesh(core_axis_name="c", subcore_axis_name="s"),
        scratch_types=[
            pltpu.VMEM((G, ROWS_W), jnp.int32),
            pltpu.VMEM((4 * ROWS_W,), jnp.float32),
            pltpu.VMEM((CH * G, 128), jnp.float32),
            pltpu.VMEM((CH * G, 128), jnp.float32),
            pltpu.VMEM((CH, C), jnp.float32),
            pltpu.SemaphoreType.DMA,
            pltpu.SemaphoreType.DMA,
        ],
    )


def kernel(feature_maps, rois):
    yy, xx, li = _sample_grid(rois)
    idx_p, w_p = _prep_call(yy, xx, li)
    idx_t = idx_p.transpose(1, 0, 2, 3).reshape(G, M)
    w_t = w_p.transpose(1, 0, 2, 3).reshape(4, M)
    idx_f = jnp.pad(idx_t, ((0, 0), (0, M_PAD - M))).reshape(-1)
    w_f = jnp.pad(w_t, ((0, 0), (0, M_PAD - M))).reshape(-1)
    table = feature_maps.reshape(B * L * H * W * HC, 128)
    out = _sc_call()(table, idx_f, w_f)
    return _repack_call(out)


# speedup vs baseline: 1.5279x; 1.1927x over previous
import functools

import jax
import jax.numpy as jnp
from jax import lax
from jax.experimental import pallas as pl
from jax.experimental.pallas import tpu as pltpu
from jax.experimental.pallas import tpu_sc as plsc

B, L, H, W, C = 2, 4, 128, 128, 256
N = 1000
CS = 7
M = B * N * CS * CS
LANES = 16
NC, NS = 2, 16
NW = NC * NS
CH = 16
TOT_CH = M // CH
CPW = -(-TOT_CH // NW)
ROWS_W = CPW * CH
M_PAD = NW * ROWS_W
HC = C // 128
G = 4 * HC

_STRIDES = (4.0, 8.0, 16.0, 32.0)


def _sample_grid(rois):
    x1, y1, x2, y2 = rois[..., 0], rois[..., 1], rois[..., 2], rois[..., 3]
    roi_area = (y2 - y1) * (x2 - x1)
    lv = (jnp.log(jnp.sqrt(jnp.maximum(roi_area, 1e-12)) / 224.0)
          / jnp.log(2.0) + 4.0)
    li = jnp.clip(jnp.round(lv).astype(jnp.int32), 2, 5) - 2
    strides = jnp.asarray(_STRIDES, dtype=jnp.float32)[li]
    Hf = jnp.float32(H)
    Wf = jnp.float32(W)
    nx1 = x1 / strides / Wf
    ny1 = y1 / strides / Hf
    nx2 = x2 / strides / Wf
    ny2 = y2 / strides / Hf
    i = jnp.arange(CS, dtype=jnp.float32)
    ys = (ny1[..., None] * (Hf - 1.0)
          + i[None, None, :] * ((ny2 - ny1)[..., None] * (Hf - 1.0) / (CS - 1)))
    xs = (nx1[..., None] * (Wf - 1.0)
          + i[None, None, :] * ((nx2 - nx1)[..., None] * (Wf - 1.0) / (CS - 1)))
    yy = jnp.broadcast_to(ys[:, :, :, None], (B, N, CS, CS))
    xx = jnp.broadcast_to(xs[:, :, None, :], (B, N, CS, CS))
    yy = yy.reshape(B, N, CS * CS)
    xx = xx.reshape(B, N, CS * CS)
    return yy, xx, li[:, :, None]


def _prep_body(yy_ref, xx_ref, li_ref, idx_ref, w_ref):
    Hf = jnp.float32(H)
    Wf = jnp.float32(W)
    for b in range(B):
        yy = yy_ref[b]
        xx = xx_ref[b]
        li = li_ref[b]
        valid = (yy >= 0.0) & (yy <= Hf - 1.0) & (xx >= 0.0) & (xx <= Wf - 1.0)
        y0 = jnp.floor(yy)
        x0 = jnp.floor(xx)
        wy = yy - y0
        wx = xx - x0
        y0i = jnp.clip(y0.astype(jnp.int32), 0, H - 1)
        x0i = jnp.clip(x0.astype(jnp.int32), 0, W - 1)
        y1i = jnp.minimum(y0i + 1, H - 1)
        x1i = jnp.minimum(x0i + 1, W - 1)
        base = li * (H * W) + b * (L * H * W)
        r0 = base + y0i * W
        r1 = base + y1i * W
        vf = valid.astype(jnp.float32)
        wx0 = 1.0 - wx
        wy0 = 1.0 - wy
        for k, rid in enumerate((r0 + x0i, r0 + x1i, r1 + x0i, r1 + x1i)):
            idx_ref[b, k] = rid
        w_ref[b, 0] = wx0 * wy0 * vf
        w_ref[b, 1] = wx * wy0 * vf
        w_ref[b, 2] = wx0 * wy * vf
        w_ref[b, 3] = wx * wy * vf


_prep_call = pl.pallas_call(
    _prep_body,
    out_shape=(
        jax.ShapeDtypeStruct((B, 4, N, CS * CS), jnp.int32),
        jax.ShapeDtypeStruct((B, 4, N, CS * CS), jnp.float32),
    ),
)


NB = 8


def _repack_body(in_ref, out_ref):
    out_ref[...] = in_ref[...].reshape(1, NB, CS, CS, C)


_repack_call = pl.pallas_call(
    _repack_body,
    grid=(B, N // NB),
    in_specs=[pl.BlockSpec((NB * CS * CS, C),
                           lambda b, nb: (b * (N // NB) + nb, 0))],
    out_specs=pl.BlockSpec((1, NB, CS, CS, C),
                           lambda b, nb: (b, nb, 0, 0, 0)),
    out_shape=jax.ShapeDtypeStruct((B, N, CS, CS, C), jnp.float32),
)


def _sc_body(table, idxf, wf, out, idx_v, w_v,
             gbuf0, gbuf1, obuf, gsem0, gsem1):
    cid = lax.axis_index("c")
    sid = lax.axis_index("s")
    wid = sid * NC + cid
    base = wid * ROWS_W
    nch = jnp.minimum(CPW, TOT_CH - CPW * wid)
    for k in range(4):
        pltpu.sync_copy(idxf.at[pl.ds(k * M_PAD + base, ROWS_W)],
                        idx_v.at[k])
    for k in range(4):
        pltpu.sync_copy(wf.at[pl.ds(k * M_PAD + base, ROWS_W)],
                        w_v.at[pl.ds(k * ROWS_W, ROWS_W)])

    gbufs = (gbuf0, gbuf1)
    gsems = (gsem0, gsem1)

    def gather_copies(c, b):
        return [pltpu.make_async_copy(
            table.at[idx_v.at[k, pl.ds(c * CH, CH)]],
            gbufs[b].at[pl.ds(k * CH, CH)], gsems[b]) for k in range(4)]

    @pl.when(nch > 0)
    def _():
        for cp in gather_copies(0, 0):
            cp.start()

    def pair(g, carry):
        for b in range(2):
            c = 2 * g + b

            @pl.when(c < nch)
            def _(c=c, b=b):
                @pl.when(c + 1 < nch)
                def _():
                    for cp in gather_copies(c + 1, 1 - b):
                        cp.start()

                gbuf = gbufs[b]
                for cp in gather_copies(c, b):
                    cp.wait()

                wchunk = [w_v[pl.ds(k * ROWS_W + c * CH, LANES)]
                          for k in range(4)]

                def row(r, carry2):
                    lane = jnp.full((LANES,), r, jnp.int32)
                    wvs = [wc.at[lane].get(mode="promise_in_bounds")
                           for wc in wchunk]

                    def vec(v, carry3):
                        col = pl.ds(v * LANES, LANES)
                        acc = gbuf[r, col] * wvs[0]
                        for k in range(1, 4):
                            acc = acc + gbuf[k * CH + r, col] * wvs[k]
                        obuf[r, col] = acc
                        return carry3

                    lax.fori_loop(0, C // LANES, vec, 0)
                    return carry2

                lax.fori_loop(0, CH, row, 0)
                pltpu.sync_copy(obuf, out.at[pl.ds(base + c * CH, CH)])
        return carry

    lax.fori_loop(0, (CPW + 1) // 2, pair, 0)


@functools.cache
def _sc_call():
    return pl.kernel(
        _sc_body,
        out_type=jax.ShapeDtypeStruct((M, C), jnp.float32),
        mesh=plsc.VectorSubcoreMesh(core_axis_name="c", subcore_axis_name="s"),
        scratch_types=[
            pltpu.VMEM((4, ROWS_W), jnp.int32),
            pltpu.VMEM((4 * ROWS_W,), jnp.float32),
            pltpu.VMEM((CH * 4, C), jnp.float32),
            pltpu.VMEM((CH * 4, C), jnp.float32),
            pltpu.VMEM((CH, C), jnp.float32),
            pltpu.SemaphoreType.DMA,
            pltpu.SemaphoreType.DMA,
        ],
    )


def kernel(feature_maps, rois):
    yy, xx, li = _sample_grid(rois)
    idx_p, w_p = _prep_call(yy, xx, li)
    idx_t = idx_p.transpose(1, 0, 2, 3).reshape(4, M)
    w_t = w_p.transpose(1, 0, 2, 3).reshape(4, M)
    idx_f = jnp.pad(idx_t, ((0, 0), (0, M_PAD - M))).reshape(-1)
    w_f = jnp.pad(w_t, ((0, 0), (0, M_PAD - M))).reshape(-1)
    table = feature_maps.reshape(B * L * H * W, C)
    out = _sc_call()(table, idx_f, w_f)
    return _repack_call(out)


# speedup vs baseline: 1.8470x; 1.2089x over previous
import functools

import jax
import jax.numpy as jnp
from jax import lax
from jax.experimental import pallas as pl
from jax.experimental.pallas import tpu as pltpu
from jax.experimental.pallas import tpu_sc as plsc

B, L, H, W, C = 2, 4, 128, 128, 256
N = 1000
CS = 7
M = B * N * CS * CS
LANES = 16
NC, NS = 2, 16
NW = NC * NS
CH = 16
TOT_CH = M // CH
CPW = -(-TOT_CH // NW)
ROWS_W = CPW * CH
M_PAD = NW * ROWS_W
HC = C // 128
G = 4 * HC

_STRIDES = (4.0, 8.0, 16.0, 32.0)


def _sample_grid(rois):
    x1, y1, x2, y2 = rois[..., 0], rois[..., 1], rois[..., 2], rois[..., 3]
    roi_area = (y2 - y1) * (x2 - x1)
    lv = (jnp.log(jnp.sqrt(jnp.maximum(roi_area, 1e-12)) / 224.0)
          / jnp.log(2.0) + 4.0)
    li = jnp.clip(jnp.round(lv).astype(jnp.int32), 2, 5) - 2
    strides = jnp.asarray(_STRIDES, dtype=jnp.float32)[li]
    Hf = jnp.float32(H)
    Wf = jnp.float32(W)
    nx1 = x1 / strides / Wf
    ny1 = y1 / strides / Hf
    nx2 = x2 / strides / Wf
    ny2 = y2 / strides / Hf
    i = jnp.arange(CS, dtype=jnp.float32)
    ys = (ny1[..., None] * (Hf - 1.0)
          + i[None, None, :] * ((ny2 - ny1)[..., None] * (Hf - 1.0) / (CS - 1)))
    xs = (nx1[..., None] * (Wf - 1.0)
          + i[None, None, :] * ((nx2 - nx1)[..., None] * (Wf - 1.0) / (CS - 1)))
    yy = jnp.broadcast_to(ys[:, :, :, None], (B, N, CS, CS))
    xx = jnp.broadcast_to(xs[:, :, None, :], (B, N, CS, CS))
    yy = yy.reshape(B, N, CS * CS)
    xx = xx.reshape(B, N, CS * CS)
    return yy, xx, li[:, :, None]


def _prep_body(yy_ref, xx_ref, li_ref, idx_ref, w_ref):
    Hf = jnp.float32(H)
    Wf = jnp.float32(W)
    for b in range(B):
        yy = yy_ref[b]
        xx = xx_ref[b]
        li = li_ref[b]
        valid = (yy >= 0.0) & (yy <= Hf - 1.0) & (xx >= 0.0) & (xx <= Wf - 1.0)
        y0 = jnp.floor(yy)
        x0 = jnp.floor(xx)
        wy = yy - y0
        wx = xx - x0
        y0i = jnp.clip(y0.astype(jnp.int32), 0, H - 1)
        x0i = jnp.clip(x0.astype(jnp.int32), 0, W - 1)
        y1i = jnp.minimum(y0i + 1, H - 1)
        x1i = jnp.minimum(x0i + 1, W - 1)
        base = li * (H * W) + b * (L * H * W)
        r0 = base + y0i * W
        r1 = base + y1i * W
        vf = valid.astype(jnp.float32)
        wx0 = 1.0 - wx
        wy0 = 1.0 - wy
        for k, rid in enumerate((r0 + x0i, r0 + x1i, r1 + x0i, r1 + x1i)):
            idx_ref[b, k] = rid
        w_ref[b, 0] = wx0 * wy0 * vf
        w_ref[b, 1] = wx * wy0 * vf
        w_ref[b, 2] = wx0 * wy * vf
        w_ref[b, 3] = wx * wy * vf


_prep_call = pl.pallas_call(
    _prep_body,
    out_shape=(
        jax.ShapeDtypeStruct((B, 4, N, CS * CS), jnp.int32),
        jax.ShapeDtypeStruct((B, 4, N, CS * CS), jnp.float32),
    ),
)


NB = 40


def _repack_body(in_ref, out_ref):
    out_ref[...] = in_ref[...].reshape(1, NB, CS, CS, C)


_repack_call = pl.pallas_call(
    _repack_body,
    grid=(B, N // NB),
    in_specs=[pl.BlockSpec((NB * CS * CS, C),
                           lambda b, nb: (b * (N // NB) + nb, 0))],
    out_specs=pl.BlockSpec((1, NB, CS, CS, C),
                           lambda b, nb: (b, nb, 0, 0, 0)),
    out_shape=jax.ShapeDtypeStruct((B, N, CS, CS, C), jnp.float32),
)


def _sc_body(table, idxf, wf, out, idx_v, w_v,
             gbuf0, gbuf1, obuf0, obuf1, gsem0, gsem1, osem0, osem1):
    cid = lax.axis_index("c")
    sid = lax.axis_index("s")
    wid = sid * NC + cid
    base = wid * ROWS_W
    nch = jnp.minimum(CPW, TOT_CH - CPW * wid)
    for k in range(4):
        pltpu.sync_copy(idxf.at[pl.ds(k * M_PAD + base, ROWS_W)],
                        idx_v.at[k])
    for k in range(4):
        pltpu.sync_copy(wf.at[pl.ds(k * M_PAD + base, ROWS_W)],
                        w_v.at[pl.ds(k * ROWS_W, ROWS_W)])

    gbufs = (gbuf0, gbuf1)
    gsems = (gsem0, gsem1)
    obufs = (obuf0, obuf1)
    osems = (osem0, osem1)

    def out_copy(c, b):
        return pltpu.make_async_copy(
            obufs[b], out.at[pl.ds(base + c * CH, CH)], osems[b])

    def gather_copies(c, b):
        return [pltpu.make_async_copy(
            table.at[idx_v.at[k, pl.ds(c * CH, CH)]],
            gbufs[b].at[pl.ds(k * CH, CH)], gsems[b]) for k in range(4)]

    @pl.when(nch > 0)
    def _():
        for cp in gather_copies(0, 0):
            cp.start()

    def pair(g, carry):
        for b in range(2):
            c = 2 * g + b

            @pl.when(c < nch)
            def _(c=c, b=b):
                @pl.when(c + 1 < nch)
                def _():
                    for cp in gather_copies(c + 1, 1 - b):
                        cp.start()

                gbuf = gbufs[b]
                obuf = obufs[b]
                for cp in gather_copies(c, b):
                    cp.wait()

                @pl.when(c >= 2)
                def _():
                    out_copy(c - 2, b).wait()

                wchunk = [w_v[pl.ds(k * ROWS_W + c * CH, LANES)]
                          for k in range(4)]

                def row(r, carry2):
                    lane = jnp.full((LANES,), r, jnp.int32)
                    wvs = [wc.at[lane].get(mode="promise_in_bounds")
                           for wc in wchunk]

                    def vec(v, carry3):
                        col = pl.ds(v * LANES, LANES)
                        acc = gbuf[r, col] * wvs[0]
                        for k in range(1, 4):
                            acc = acc + gbuf[k * CH + r, col] * wvs[k]
                        obuf[r, col] = acc
                        return carry3

                    lax.fori_loop(0, C // LANES, vec, 0)
                    return carry2

                lax.fori_loop(0, CH, row, 0)
                out_copy(c, b).start()
        return carry

    lax.fori_loop(0, (CPW + 1) // 2, pair, 0)

    for b in range(2):
        @pl.when(nch > b)
        def _(b=b):
            out_copy(0, b).wait()


@functools.cache
def _sc_call():
    return pl.kernel(
        _sc_body,
        out_type=jax.ShapeDtypeStruct((M, C), jnp.float32),
        mesh=plsc.VectorSubcoreMesh(core_axis_name="c", subcore_axis_name="s"),
        scratch_types=[
            pltpu.VMEM((4, ROWS_W), jnp.int32),
            pltpu.VMEM((4 * ROWS_W,), jnp.float32),
            pltpu.VMEM((CH * 4, C), jnp.float32),
            pltpu.VMEM((CH * 4, C), jnp.float32),
            pltpu.VMEM((CH, C), jnp.float32),
            pltpu.VMEM((CH, C), jnp.float32),
            pltpu.SemaphoreType.DMA,
            pltpu.SemaphoreType.DMA,
            pltpu.SemaphoreType.DMA,
            pltpu.SemaphoreType.DMA,
        ],
    )


def kernel(feature_maps, rois):
    yy, xx, li = _sample_grid(rois)
    idx_p, w_p = _prep_call(yy, xx, li)
    idx_t = idx_p.transpose(1, 0, 2, 3).reshape(4, M)
    w_t = w_p.transpose(1, 0, 2, 3).reshape(4, M)
    idx_f = jnp.pad(idx_t, ((0, 0), (0, M_PAD - M))).reshape(-1)
    w_f = jnp.pad(w_t, ((0, 0), (0, M_PAD - M))).reshape(-1)
    table = feature_maps.reshape(B * L * H * W, C)
    out = _sc_call()(table, idx_f, w_f)
    return _repack_call(out)
